# baseline (speedup 1.0000x reference)
import jax
import jax.numpy as jnp
from jax.experimental import pallas as pl
from jax.experimental.pallas import tpu as pltpu

_B = 16384
_TB = 2048
_C = [11, 2, 7, 2, 2]
_OFF = [5, 16, 18, 25, 27]
_F = 29


def _mlp_body(client_ref, x_ref, w1_ref, b1_ref, w2_ref, b2_ref,
              w3_ref, b3_ref, w4_ref, b4_ref, w5_ref, b5_ref, out_ref):
    cl = client_ref[0]
    xb = x_ref[...]

    lane = jax.lax.broadcasted_iota(jnp.int32, (1, 10), 1)
    cvec = jnp.full((1, 10), 1, jnp.int32)
    ovec = jnp.zeros((1, 10), jnp.int32)
    for i in range(5):
        cvec = jnp.where(lane == 5 + i, _C[i], cvec)
        ovec = jnp.where(lane == 5 + i, _OFF[i] - 5, ovec)
    z = (xb - cl).astype(jnp.int32)
    z = jnp.where(z < 0, z + cvec, z)
    valid = (lane >= 5) & (z >= 0) & (z < cvec)
    bits = jnp.where(valid, jnp.int32(1) << (z + ovec), 0)
    mask = (bits[:, 5:6] + bits[:, 6:7] + bits[:, 7:8]
            + bits[:, 8:9] + bits[:, 9:10])
    colloc = jax.lax.broadcasted_iota(jnp.int32, (_TB, _F - 5), 1)
    hitf = ((mask >> colloc) & 1).astype(jnp.float32)
    feats = jnp.concatenate([xb[:, 0:5], hitf], axis=1)

    h = jnp.tanh(jnp.dot(feats, w1_ref[...],
                         preferred_element_type=jnp.float32) + b1_ref[...])
    h = jnp.tanh(jnp.dot(h, w2_ref[...],
                         preferred_element_type=jnp.float32) + b2_ref[...])
    h = jnp.tanh(jnp.dot(h, w3_ref[...],
                         preferred_element_type=jnp.float32) + b3_ref[...])
    h = jnp.tanh(jnp.dot(h, w4_ref[...],
                         preferred_element_type=jnp.float32) + b4_ref[...])
    out_ref[...] = jnp.dot(h, w5_ref[...],
                           preferred_element_type=jnp.float32) + b5_ref[...]


def kernel(x, client, W1, b1, W2, b2, W3, b3, W4, b4, W5, b5):
    n = x.shape[0]
    cl = jnp.asarray(client, jnp.float32).reshape(1)
    full = lambda s: pl.BlockSpec(s, lambda i: (0, 0))
    grid = (n // _TB,)
    return pl.pallas_call(
        _mlp_body,
        grid=grid,
        in_specs=[
            pl.BlockSpec(memory_space=pltpu.SMEM),
            pl.BlockSpec((_TB, 10), lambda i: (i, 0)),
            full(W1.shape), full((1, b1.shape[0])),
            full(W2.shape), full((1, b2.shape[0])),
            full(W3.shape), full((1, b3.shape[0])),
            full(W4.shape), full((1, b4.shape[0])),
            full(W5.shape), full((1, b5.shape[0])),
        ],
        out_specs=pl.BlockSpec((_TB, 64), lambda i: (i, 0)),
        out_shape=jax.ShapeDtypeStruct((n, 64), jnp.float32),
        compiler_params=pltpu.CompilerParams(
            dimension_semantics=("parallel",)),
    )(cl, x, W1, b1.reshape(1, -1), W2, b2.reshape(1, -1),
      W3, b3.reshape(1, -1), W4, b4.reshape(1, -1), W5, b5.reshape(1, -1))


# speedup vs baseline: 1.0326x; 1.0326x over previous
import jax
import jax.numpy as jnp
from jax.experimental import pallas as pl
from jax.experimental.pallas import tpu as pltpu

_B = 16384
_TB = 4096
_C = [11, 2, 7, 2, 2]
_OFF = [5, 16, 18, 25, 27]
_F = 29


def _mlp_body(client_ref, x_ref, w1_ref, b1_ref, w2_ref, b2_ref,
              w3_ref, b3_ref, w4_ref, b4_ref, w5_ref, b5_ref, out_ref):
    cl = client_ref[0]
    xb = x_ref[...]

    lane = jax.lax.broadcasted_iota(jnp.int32, (1, 10), 1)
    cvec = jnp.full((1, 10), 1, jnp.int32)
    ovec = jnp.zeros((1, 10), jnp.int32)
    for i in range(5):
        cvec = jnp.where(lane == 5 + i, _C[i], cvec)
        ovec = jnp.where(lane == 5 + i, _OFF[i] - 5, ovec)
    z = (xb - cl).astype(jnp.int32)
    z = jnp.where(z < 0, z + cvec, z)
    valid = (lane >= 5) & (z >= 0) & (z < cvec)
    bits = jnp.where(valid, jnp.int32(1) << (z + ovec), 0)
    mask = (bits[:, 5:6] + bits[:, 6:7] + bits[:, 7:8]
            + bits[:, 8:9] + bits[:, 9:10])
    colloc = jax.lax.broadcasted_iota(jnp.int32, (_TB, _F - 5), 1)
    hitf = ((mask >> colloc) & 1).astype(jnp.float32)
    feats = jnp.concatenate([xb[:, 0:5], hitf], axis=1)

    h = jnp.tanh(jnp.dot(feats, w1_ref[...],
                         preferred_element_type=jnp.float32) + b1_ref[...])
    h = jnp.tanh(jnp.dot(h, w2_ref[...],
                         preferred_element_type=jnp.float32) + b2_ref[...])
    h = jnp.tanh(jnp.dot(h, w3_ref[...],
                         preferred_element_type=jnp.float32) + b3_ref[...])
    h = jnp.tanh(jnp.dot(h, w4_ref[...],
                         preferred_element_type=jnp.float32) + b4_ref[...])
    out_ref[...] = jnp.dot(h, w5_ref[...],
                           preferred_element_type=jnp.float32) + b5_ref[...]


def kernel(x, client, W1, b1, W2, b2, W3, b3, W4, b4, W5, b5):
    n = x.shape[0]
    cl = jnp.asarray(client, jnp.float32).reshape(1)
    full = lambda s: pl.BlockSpec(s, lambda i: (0, 0))
    grid = (n // _TB,)
    return pl.pallas_call(
        _mlp_body,
        grid=grid,
        in_specs=[
            pl.BlockSpec(memory_space=pltpu.SMEM),
            pl.BlockSpec((_TB, 10), lambda i: (i, 0)),
            full(W1.shape), full((1, b1.shape[0])),
            full(W2.shape), full((1, b2.shape[0])),
            full(W3.shape), full((1, b3.shape[0])),
            full(W4.shape), full((1, b4.shape[0])),
            full(W5.shape), full((1, b5.shape[0])),
        ],
        out_specs=pl.BlockSpec((_TB, 64), lambda i: (i, 0)),
        out_shape=jax.ShapeDtypeStruct((n, 64), jnp.float32),
        compiler_params=pltpu.CompilerParams(
            dimension_semantics=("parallel",)),
    )(cl, x, W1, b1.reshape(1, -1), W2, b2.reshape(1, -1),
      W3, b3.reshape(1, -1), W4, b4.reshape(1, -1), W5, b5.reshape(1, -1))
